# baseline (speedup 1.0000x reference)
import functools

import jax
import jax.numpy as jnp
from jax import lax
from jax.experimental import pallas as pl
from jax.experimental.pallas import tpu as pltpu
from jax.experimental.pallas import tpu_sc as plsc

BATCH = 4096
SEQ = 200
HIDDEN = 64
N = BATCH * SEQ

_info = plsc.get_sparse_core_info()
NC, NS = _info.num_cores, _info.num_subcores
NW = NC * NS
PER_W = N // NW
CHUNK = 128
N_CHUNKS = PER_W // CHUNK

_mesh = plsc.VectorSubcoreMesh(core_axis_name="c", subcore_axis_name="s")


@functools.partial(
    pl.kernel,
    out_type=jax.ShapeDtypeStruct((N, HIDDEN), jnp.float32),
    mesh=_mesh,
    scratch_types=[
        pltpu.VMEM((CHUNK,), jnp.int32),
        pltpu.VMEM((CHUNK, HIDDEN), jnp.float32),
        pltpu.SemaphoreType.DMA,
    ],
    compiler_params=pltpu.CompilerParams(use_tc_tiling_on_sc=False),
)
def _gather_kernel(ids_hbm, tab_hbm, out_hbm, idx_v, rows_v, sem):
    wid = lax.axis_index("s") * NC + lax.axis_index("c")
    base = wid * PER_W

    def step(i, carry):
        off = base + i * CHUNK
        pltpu.sync_copy(ids_hbm.at[pl.ds(off, CHUNK)], idx_v)
        pltpu.async_copy(tab_hbm.at[idx_v], rows_v, sem).wait()
        pltpu.sync_copy(rows_v, out_hbm.at[pl.ds(off, CHUNK)])
        return carry

    lax.fori_loop(0, N_CHUNKS, step, 0)


def kernel(input_ids, word_embeddings):
    flat_ids = input_ids.reshape(N).astype(jnp.int32)
    out = _gather_kernel(flat_ids, word_embeddings)
    return out.reshape(BATCH, SEQ, HIDDEN)


# speedup vs baseline: 1.1902x; 1.1902x over previous
import functools

import jax
import jax.numpy as jnp
from jax import lax
from jax.experimental import pallas as pl
from jax.experimental.pallas import tpu as pltpu
from jax.experimental.pallas import tpu_sc as plsc

BATCH = 4096
SEQ = 200
HIDDEN = 64
N = BATCH * SEQ

_info = plsc.get_sparse_core_info()
NC, NS = _info.num_cores, _info.num_subcores
NW = NC * NS
PER_W = N // NW
CHUNK = 128
N_CHUNKS = PER_W // CHUNK
NBUF = 4

_mesh = plsc.VectorSubcoreMesh(core_axis_name="c", subcore_axis_name="s")


@functools.partial(
    pl.kernel,
    out_type=jax.ShapeDtypeStruct((N, HIDDEN), jnp.float32),
    mesh=_mesh,
    scratch_types=[
        pltpu.VMEM((NBUF, CHUNK), jnp.int32),
        pltpu.VMEM((NBUF, CHUNK, HIDDEN), jnp.float32),
        pltpu.SemaphoreType.DMA((NBUF,)),
        pltpu.SemaphoreType.DMA((NBUF,)),
        pltpu.SemaphoreType.DMA((NBUF,)),
    ],
    compiler_params=pltpu.CompilerParams(use_tc_tiling_on_sc=False),
)
def _gather_kernel(ids_hbm, tab_hbm, out_hbm, idx_v, rows_v, sem_i, sem_g,
                   sem_o):
    wid = lax.axis_index("s") * NC + lax.axis_index("c")
    base = wid * PER_W

    def idx_copy(chunk, slot):
        return pltpu.make_async_copy(
            ids_hbm.at[pl.ds(base + chunk * CHUNK, CHUNK)],
            idx_v.at[slot], sem_i.at[slot])

    def out_copy(chunk, slot):
        return pltpu.make_async_copy(
            rows_v.at[slot],
            out_hbm.at[pl.ds(base + chunk * CHUNK, CHUNK)], sem_o.at[slot])

    for b in range(NBUF):
        idx_copy(b, b).start()

    def group(g, carry):
        for b in range(NBUF):
            c = g * NBUF + b
            idx_copy(c, b).wait()
            pl.when(g > 0)(lambda b=b: out_copy(0, b).wait())
            pltpu.make_async_copy(tab_hbm.at[idx_v.at[b]], rows_v.at[b],
                                  sem_g.at[b]).start()
            bp = (b - 1) % NBUF

            def finish(c=c, b=b, bp=bp):
                pltpu.make_async_copy(tab_hbm.at[idx_v.at[bp]],
                                      rows_v.at[bp], sem_g.at[bp]).wait()
                out_copy(c - 1, bp).start()
                pl.when(c + NBUF - 1 < N_CHUNKS)(
                    lambda: idx_copy(c + NBUF - 1, bp).start())

            if b == 0:
                pl.when(g > 0)(finish)
            else:
                finish()
        return carry

    lax.fori_loop(0, N_CHUNKS // NBUF, group, 0, unroll=False)

    bl = (N_CHUNKS - 1) % NBUF
    pltpu.make_async_copy(tab_hbm.at[idx_v.at[bl]], rows_v.at[bl],
                          sem_g.at[bl]).wait()
    out_copy(N_CHUNKS - 1, bl).start()
    for b in range(NBUF):
        out_copy(0, b).wait()


def kernel(input_ids, word_embeddings):
    flat_ids = input_ids.reshape(N).astype(jnp.int32)
    out = _gather_kernel(flat_ids, word_embeddings)
    return out.reshape(BATCH, SEQ, HIDDEN)


# speedup vs baseline: 1.1944x; 1.0035x over previous
import functools

import jax
import jax.numpy as jnp
from jax import lax
from jax.experimental import pallas as pl
from jax.experimental.pallas import tpu as pltpu
from jax.experimental.pallas import tpu_sc as plsc

BATCH = 4096
SEQ = 200
HIDDEN = 64
N = BATCH * SEQ
HEAD = 128
TAIL = SEQ - HEAD

_info = plsc.get_sparse_core_info()
NC, NS = _info.num_cores, _info.num_subcores
NW = NC * NS

ROWS_PER_W = BATCH // NW
A_ITERS = ROWS_PER_W // 8
NBUF = 4

_mesh = plsc.VectorSubcoreMesh(core_axis_name="c", subcore_axis_name="s")


@functools.partial(
    pl.kernel,
    out_type=jax.ShapeDtypeStruct((2 * BATCH, HEAD), jnp.int32),
    mesh=_mesh,
    scratch_types=[
        pltpu.SemaphoreType.DMA,
    ],
)
def _repack_ids(ids_hbm, out_hbm, sem):
    wid = lax.axis_index("s") * NC + lax.axis_index("c")
    row0 = wid * ROWS_PER_W
    head = pltpu.make_async_copy(
        ids_hbm.at[pl.ds(row0, ROWS_PER_W), pl.ds(0, HEAD)],
        out_hbm.at[pl.ds(row0, ROWS_PER_W), :], sem)
    tail = pltpu.make_async_copy(
        ids_hbm.at[pl.ds(row0, ROWS_PER_W), pl.ds(HEAD, HEAD)],
        out_hbm.at[pl.ds(BATCH + row0, ROWS_PER_W), :], sem)
    head.start()
    tail.start()
    head.wait()
    tail.wait()


@functools.partial(
    pl.kernel,
    out_type=jax.ShapeDtypeStruct((N, HIDDEN), jnp.float32),
    mesh=_mesh,
    scratch_types=[
        pltpu.VMEM((NBUF, 2 * HEAD), jnp.int32),
        pltpu.VMEM((NBUF, SEQ, HIDDEN), jnp.float32),
        pltpu.SemaphoreType.DMA((NBUF,)),
        pltpu.SemaphoreType.DMA((NBUF,)),
        pltpu.SemaphoreType.DMA((NBUF,)),
    ],
    compiler_params=pltpu.CompilerParams(use_tc_tiling_on_sc=False),
)
def _gather_kernel(ids_hbm, tab_hbm, out_hbm, idx_v, rows_v, sem_i, sem_g,
                   sem_o):
    wid = lax.axis_index("s") * NC + lax.axis_index("c")
    base = wid * ROWS_PER_W

    def idx_copies(c, slot):
        return [
            pltpu.make_async_copy(ids_hbm.at[base + c],
                                  idx_v.at[slot, pl.ds(0, HEAD)],
                                  sem_i.at[slot]),
            pltpu.make_async_copy(ids_hbm.at[BATCH + base + c],
                                  idx_v.at[slot, pl.ds(HEAD, HEAD)],
                                  sem_i.at[slot]),
        ]

    def gathers(c, slot):
        return [
            pltpu.make_async_copy(
                tab_hbm.at[idx_v.at[slot, pl.ds(0, HEAD)]],
                rows_v.at[slot, pl.ds(0, HEAD), :], sem_g.at[slot]),
            pltpu.make_async_copy(
                tab_hbm.at[idx_v.at[slot, pl.ds(HEAD, TAIL)]],
                rows_v.at[slot, pl.ds(HEAD, TAIL), :], sem_g.at[slot]),
        ]

    def out_copy(c, slot):
        return pltpu.make_async_copy(
            rows_v.at[slot],
            out_hbm.at[pl.ds((base + c) * SEQ, SEQ)], sem_o.at[slot])

    for b in range(NBUF):
        for cp in idx_copies(b, b):
            cp.start()

    def group(g, carry):
        for b in range(NBUF):
            c = g * NBUF + b
            for cp in idx_copies(c, b):
                cp.wait()
            pl.when(g > 0)(lambda b=b: out_copy(0, b).wait())
            for cp in gathers(c, b):
                cp.start()
            bp = (b - 1) % NBUF

            def finish(c=c, b=b, bp=bp):
                for cp in gathers(c - 1, bp):
                    cp.wait()
                out_copy(c - 1, bp).start()

                def prefetch(c=c, bp=bp):
                    for cp in idx_copies(c + NBUF - 1, bp):
                        cp.start()

                pl.when(c + NBUF - 1 < ROWS_PER_W)(prefetch)

            if b == 0:
                pl.when(g > 0)(finish)
            else:
                finish()
        return carry

    lax.fori_loop(0, ROWS_PER_W // NBUF, group, 0, unroll=False)

    bl = (ROWS_PER_W - 1) % NBUF
    for cp in gathers(ROWS_PER_W - 1, bl):
        cp.wait()
    out_copy(ROWS_PER_W - 1, bl).start()
    for b in range(NBUF):
        out_copy(0, b).wait()


def kernel(input_ids, word_embeddings):
    ids2 = jnp.pad(input_ids.astype(jnp.int32), ((0, 0), (0, 2 * HEAD - SEQ)))
    packed = _repack_ids(ids2)
    out = _gather_kernel(packed, word_embeddings)
    return out.reshape(BATCH, SEQ, HIDDEN)
